# baseline (speedup 1.0000x reference)
import functools

import jax
import jax.numpy as jnp
from jax.experimental import pallas as pl
from jax.experimental.pallas import tpu as pltpu

BN_EPS = 1e-5
LANE = 128
SUBLANE = 8
VMEM_LIMIT = 60 * 1024 * 1024


def _stats_kernel(x_ref, w_ref, gamma_ref, beta_ref, ab_ref, acc_ref, *,
                  spb, nb0, rows_per_slab, crop_rows, pad_rows,
                  h_lo, h_hi, inv_cnt):
    b = pl.program_id(0)

    @pl.when(b == 0)
    def _():
        acc_ref[...] = jnp.zeros_like(acc_ref)

    v = x_ref[...].reshape(spb, rows_per_slab, LANE)
    vc = v[:, pad_rows:pad_rows + crop_rows, :]
    gh = b * spb + jax.lax.broadcasted_iota(jnp.int32, (spb, 1, 1), 0)
    mask = ((gh >= h_lo) & (gh < h_hi)).astype(jnp.float32)
    vm = (vc * mask).reshape(-1, SUBLANE, LANE)
    acc_ref[0] += jnp.sum(vm, axis=0)
    acc_ref[1] += jnp.sum(vm * vm, axis=0)

    @pl.when(b == nb0 - 1)
    def _finalize():
        s1 = jnp.sum(acc_ref[0])
        s2 = jnp.sum(acc_ref[1])
        w = w_ref[0]
        mean_t = w * s1 * inv_cnt
        ex2_t = w * w * s2 * inv_cnt
        var = jnp.maximum(ex2_t - mean_t * mean_t, 0.0)
        a = gamma_ref[0] * jax.lax.rsqrt(var + BN_EPS)
        ab_ref[0] = w * a
        ab_ref[1] = beta_ref[0] - mean_t * a


def _apply_kernel(xc_ref, ab_ref, o_ref):
    o_ref[...] = jnp.maximum(xc_ref[...] * ab_ref[0] + ab_ref[1], 0.0)


@functools.partial(jax.jit, static_argnames=("stride", "padding"))
def _forward(x, w_t, gamma, beta, *, stride=1, padding=1):
    N, Cin, H, W = x.shape
    Cin_w, Cout, kH, kW = w_t.shape
    assert Cin == 1 and Cout == 1 and kH == 1 and kW == 1 and stride == 1

    Ho = (H - 1) * stride - 2 * padding + kH
    Wo = (W - 1) * stride - 2 * padding + kW
    assert Ho > 0 and Wo > 0
    assert N % LANE == 0 and (W * N) % LANE == 0 and (padding * N) % LANE == 0

    rows_per_slab = W * N // LANE
    crop_rows = Wo * N // LANE
    pad_rows = padding * N // LANE

    spb = 1
    for cand in (16, 13, 10, 8, 5, 4, 2):
        if H % cand == 0:
            spb = cand
            break
    nb0 = H // spb

    z = jnp.transpose(x, (2, 3, 1, 0)).reshape(H * rows_per_slab, LANE)
    w1 = w_t.reshape(1).astype(jnp.float32)
    gamma32 = gamma.astype(jnp.float32)
    beta32 = beta.astype(jnp.float32)
    inv_cnt = 1.0 / float(N * Ho * Wo)

    ab = pl.pallas_call(
        functools.partial(
            _stats_kernel, spb=spb, nb0=nb0, rows_per_slab=rows_per_slab,
            crop_rows=crop_rows, pad_rows=pad_rows,
            h_lo=padding, h_hi=padding + Ho, inv_cnt=inv_cnt),
        out_shape=jax.ShapeDtypeStruct((2,), jnp.float32),
        grid=(nb0,),
        in_specs=[
            pl.BlockSpec((spb * rows_per_slab, LANE), lambda b: (b, 0)),
            pl.BlockSpec(memory_space=pltpu.MemorySpace.SMEM),
            pl.BlockSpec(memory_space=pltpu.MemorySpace.SMEM),
            pl.BlockSpec(memory_space=pltpu.MemorySpace.SMEM),
        ],
        out_specs=pl.BlockSpec(memory_space=pltpu.MemorySpace.SMEM),
        scratch_shapes=[pltpu.VMEM((2, SUBLANE, LANE), jnp.float32)],
        compiler_params=pltpu.CompilerParams(
            dimension_semantics=("arbitrary",),
            vmem_limit_bytes=VMEM_LIMIT),
    )(z, w1, gamma32, beta32)

    xq = x[:, :, padding:padding + Ho, padding:padding + Wo]

    bn = 32
    while N % bn != 0:
        bn //= 2
    out = pl.pallas_call(
        _apply_kernel,
        out_shape=jax.ShapeDtypeStruct((N, Cout, Ho, Wo), x.dtype),
        grid=(N // bn,),
        in_specs=[
            pl.BlockSpec((bn, 1, Ho, Wo), lambda b: (b, 0, 0, 0)),
            pl.BlockSpec(memory_space=pltpu.MemorySpace.SMEM),
        ],
        out_specs=pl.BlockSpec((bn, 1, Ho, Wo), lambda b: (b, 0, 0, 0)),
        compiler_params=pltpu.CompilerParams(
            dimension_semantics=("arbitrary",),
            vmem_limit_bytes=VMEM_LIMIT),
    )(xq, ab)
    return out


def kernel(x, w_t, gamma, beta):
    return _forward(x, w_t, gamma, beta, stride=1, padding=1)


# speedup vs baseline: 2.1822x; 2.1822x over previous
import functools

import jax
import jax.numpy as jnp
from jax.experimental import pallas as pl
from jax.experimental.pallas import tpu as pltpu

BN_EPS = 1e-5
LANE = 128
SUBLANE = 8
VMEM_LIMIT = 60 * 1024 * 1024
HG = 8


def _fused_kernel(x_ref, w_ref, gamma_ref, beta_ref, o_ref,
                  xc_ref, acc_ref, ab_ref, *,
                  spb, nb0, nb1, rows_per_slab, crop_rows, pad_rows,
                  h_lo, h_hi, nlt, wo, inv_cnt):
    p = pl.program_id(0)
    b = pl.program_id(1)

    @pl.when((p == 0) & (b < nb0))
    def _phase0():
        @pl.when(b == 0)
        def _():
            acc_ref[...] = jnp.zeros_like(acc_ref)

        v = x_ref[...].reshape(spb, rows_per_slab, LANE)
        vc = v[:, pad_rows:pad_rows + crop_rows, :]
        xc_ref[pl.ds(b * spb, spb)] = vc.reshape(spb, wo, nlt, LANE)

        gh = b * spb + jax.lax.broadcasted_iota(jnp.int32, (spb, 1, 1), 0)
        mask = ((gh >= h_lo) & (gh < h_hi)).astype(jnp.float32)
        vm = vc * mask
        vm8 = vm.reshape(-1, SUBLANE, LANE)
        acc_ref[0] += jnp.sum(vm8, axis=0)
        acc_ref[1] += jnp.sum(vm8 * vm8, axis=0)

        @pl.when(b == nb0 - 1)
        def _finalize():
            s1 = jnp.sum(acc_ref[0])
            s2 = jnp.sum(acc_ref[1])
            w = w_ref[0]
            mean_t = w * s1 * inv_cnt
            ex2_t = w * w * s2 * inv_cnt
            var = jnp.maximum(ex2_t - mean_t * mean_t, 0.0)
            a = gamma_ref[0] * jax.lax.rsqrt(var + BN_EPS)
            ab_ref[0] = w * a
            ab_ref[1] = beta_ref[0] - mean_t * a

    @pl.when((p == 1) & (b < nb1))
    def _phase1():
        a = ab_ref[0]
        c = ab_ref[1]
        for i in range(HG):
            h = h_lo + b * HG + i
            col = jnp.concatenate(
                [xc_ref[h, :, nb, :].T for nb in range(nlt)], axis=0)
            o_ref[:, 0, i, :] = jnp.maximum(col * a + c, 0.0)


@functools.partial(jax.jit, static_argnames=("stride", "padding"))
def _forward(x, w_t, gamma, beta, *, stride=1, padding=1):
    N, Cin, H, W = x.shape
    Cin_w, Cout, kH, kW = w_t.shape
    assert Cin == 1 and Cout == 1 and kH == 1 and kW == 1 and stride == 1

    Ho = (H - 1) * stride - 2 * padding + kH
    Wo = (W - 1) * stride - 2 * padding + kW
    assert Ho > 0 and Wo > 0 and Ho % HG == 0
    assert N % LANE == 0 and (W * N) % LANE == 0 and (padding * N) % LANE == 0

    rows_per_slab = W * N // LANE
    crop_rows = Wo * N // LANE
    pad_rows = padding * N // LANE
    nlt = N // LANE

    spb = 1
    for cand in (16, 13, 10, 8, 5, 4, 2):
        if H % cand == 0:
            spb = cand
            break
    nb0 = H // spb
    nb1 = Ho // HG
    nsteps = max(nb0, nb1)

    z = jnp.transpose(x, (2, 3, 1, 0)).reshape(H * rows_per_slab, LANE)
    w1 = w_t.reshape(1).astype(jnp.float32)
    gamma32 = gamma.astype(jnp.float32)
    beta32 = beta.astype(jnp.float32)
    inv_cnt = 1.0 / float(N * Ho * Wo)

    return pl.pallas_call(
        functools.partial(
            _fused_kernel, spb=spb, nb0=nb0, nb1=nb1,
            rows_per_slab=rows_per_slab, crop_rows=crop_rows,
            pad_rows=pad_rows, h_lo=padding, h_hi=padding + Ho,
            nlt=nlt, wo=Wo, inv_cnt=inv_cnt),
        out_shape=jax.ShapeDtypeStruct((N, Cout, Ho, Wo), x.dtype),
        grid=(2, nsteps),
        in_specs=[
            pl.BlockSpec(
                (spb * rows_per_slab, LANE),
                lambda p, b: (jnp.where(p == 0, jnp.minimum(b, nb0 - 1),
                                        nb0 - 1), 0)),
            pl.BlockSpec(memory_space=pltpu.MemorySpace.SMEM),
            pl.BlockSpec(memory_space=pltpu.MemorySpace.SMEM),
            pl.BlockSpec(memory_space=pltpu.MemorySpace.SMEM),
        ],
        out_specs=pl.BlockSpec(
            (N, 1, HG, Wo),
            lambda p, b: (0, 0, jnp.minimum(b, nb1 - 1) * p, 0)),
        scratch_shapes=[
            pltpu.VMEM((H, Wo, nlt, LANE), jnp.float32),
            pltpu.VMEM((2, SUBLANE, LANE), jnp.float32),
            pltpu.SMEM((2,), jnp.float32),
        ],
        compiler_params=pltpu.CompilerParams(
            dimension_semantics=("arbitrary", "arbitrary"),
            vmem_limit_bytes=VMEM_LIMIT),
    )(z, w1, gamma32, beta32)


def kernel(x, w_t, gamma, beta):
    return _forward(x, w_t, gamma, beta, stride=1, padding=1)


# speedup vs baseline: 2.2314x; 1.0225x over previous
import functools

import jax
import jax.numpy as jnp
from jax.experimental import pallas as pl
from jax.experimental.pallas import tpu as pltpu

BN_EPS = 1e-5
LANE = 128
SUBLANE = 8
VMEM_LIMIT = 60 * 1024 * 1024
HG = 16


def _fused_kernel(x_ref, w_ref, gamma_ref, beta_ref, o_ref,
                  xc_ref, acc_ref, ab_ref, *,
                  spb, nb0, nb1, rows_per_slab, crop_rows, pad_rows,
                  h_lo, h_hi, nlt, wo, inv_cnt):
    p = pl.program_id(0)
    b = pl.program_id(1)

    @pl.when((p == 0) & (b < nb0))
    def _phase0():
        @pl.when(b == 0)
        def _():
            acc_ref[...] = jnp.zeros_like(acc_ref)

        v = x_ref[...].reshape(spb, rows_per_slab, LANE)
        vc = v[:, pad_rows:pad_rows + crop_rows, :]
        xc_ref[pl.ds(b * spb, spb)] = vc.reshape(spb, wo, nlt, LANE)

        gh = b * spb + jax.lax.broadcasted_iota(jnp.int32, (spb, 1, 1), 0)
        mask = ((gh >= h_lo) & (gh < h_hi)).astype(jnp.float32)
        vm = vc * mask
        vm8 = vm.reshape(-1, SUBLANE, LANE)
        acc_ref[0] += jnp.sum(vm8, axis=0)
        acc_ref[1] += jnp.sum(vm8 * vm8, axis=0)

        @pl.when(b == nb0 - 1)
        def _finalize():
            s1 = jnp.sum(acc_ref[0])
            s2 = jnp.sum(acc_ref[1])
            w = w_ref[0]
            mean_t = w * s1 * inv_cnt
            ex2_t = w * w * s2 * inv_cnt
            var = jnp.maximum(ex2_t - mean_t * mean_t, 0.0)
            a = gamma_ref[0] * jax.lax.rsqrt(var + BN_EPS)
            ab_ref[0] = w * a
            ab_ref[1] = beta_ref[0] - mean_t * a

    @pl.when((p == 1) & (b < nb1))
    def _phase1():
        a = ab_ref[0]
        c = ab_ref[1]
        for i in range(HG):
            h = h_lo + b * HG + i
            col = jnp.concatenate(
                [xc_ref[h, :, nb, :].T for nb in range(nlt)], axis=0)
            o_ref[:, 0, i, :] = jnp.maximum(col * a + c, 0.0)


@functools.partial(jax.jit, static_argnames=("stride", "padding"))
def _forward(x, w_t, gamma, beta, *, stride=1, padding=1):
    N, Cin, H, W = x.shape
    Cin_w, Cout, kH, kW = w_t.shape
    assert Cin == 1 and Cout == 1 and kH == 1 and kW == 1 and stride == 1

    Ho = (H - 1) * stride - 2 * padding + kH
    Wo = (W - 1) * stride - 2 * padding + kW
    assert Ho > 0 and Wo > 0 and Ho % HG == 0
    assert N % LANE == 0 and (W * N) % LANE == 0 and (padding * N) % LANE == 0

    rows_per_slab = W * N // LANE
    crop_rows = Wo * N // LANE
    pad_rows = padding * N // LANE
    nlt = N // LANE

    spb = 1
    for cand in (26, 16, 13, 10, 8, 5, 4, 2):
        if H % cand == 0:
            spb = cand
            break
    nb0 = H // spb
    nb1 = Ho // HG
    nsteps = max(nb0, nb1)

    z = jnp.transpose(x, (2, 3, 1, 0)).reshape(H * rows_per_slab, LANE)
    w1 = w_t.reshape(1).astype(jnp.float32)
    gamma32 = gamma.astype(jnp.float32)
    beta32 = beta.astype(jnp.float32)
    inv_cnt = 1.0 / float(N * Ho * Wo)

    return pl.pallas_call(
        functools.partial(
            _fused_kernel, spb=spb, nb0=nb0, nb1=nb1,
            rows_per_slab=rows_per_slab, crop_rows=crop_rows,
            pad_rows=pad_rows, h_lo=padding, h_hi=padding + Ho,
            nlt=nlt, wo=Wo, inv_cnt=inv_cnt),
        out_shape=jax.ShapeDtypeStruct((N, Cout, Ho, Wo), x.dtype),
        grid=(2, nsteps),
        in_specs=[
            pl.BlockSpec(
                (spb * rows_per_slab, LANE),
                lambda p, b: (jnp.where(p == 0, jnp.minimum(b, nb0 - 1),
                                        nb0 - 1), 0)),
            pl.BlockSpec(memory_space=pltpu.MemorySpace.SMEM),
            pl.BlockSpec(memory_space=pltpu.MemorySpace.SMEM),
            pl.BlockSpec(memory_space=pltpu.MemorySpace.SMEM),
        ],
        out_specs=pl.BlockSpec(
            (N, 1, HG, Wo),
            lambda p, b: (0, 0, jnp.minimum(b, nb1 - 1) * p, 0)),
        scratch_shapes=[
            pltpu.VMEM((H, Wo, nlt, LANE), jnp.float32),
            pltpu.VMEM((2, SUBLANE, LANE), jnp.float32),
            pltpu.SMEM((2,), jnp.float32),
        ],
        compiler_params=pltpu.CompilerParams(
            dimension_semantics=("arbitrary", "arbitrary"),
            vmem_limit_bytes=VMEM_LIMIT),
    )(z, w1, gamma32, beta32)


def kernel(x, w_t, gamma, beta):
    return _forward(x, w_t, gamma, beta, stride=1, padding=1)


# speedup vs baseline: 2.3013x; 1.0313x over previous
import functools

import jax
import jax.numpy as jnp
from jax.experimental import pallas as pl
from jax.experimental.pallas import tpu as pltpu

BN_EPS = 1e-5
LANE = 128
SUBLANE = 8
VMEM_LIMIT = 60 * 1024 * 1024
HG = 16


def _fused_kernel(x_ref, w_ref, gamma_ref, beta_ref, o_ref,
                  xc_ref, acc_ref, ab_ref, *,
                  spb, nb0, nb1, rows_per_slab, crop_rows, pad_rows,
                  h_lo, h_hi, nlt, wo, inv_cnt):
    p = pl.program_id(0)
    b = pl.program_id(1)

    @pl.when((p == 0) & (b < nb0))
    def _phase0():
        @pl.when(b == 0)
        def _():
            acc_ref[...] = jnp.zeros_like(acc_ref)

        v = x_ref[...].reshape(spb, rows_per_slab, LANE)
        vc = v[:, pad_rows:pad_rows + crop_rows, :]
        vc4 = vc.reshape(spb, wo, nlt, LANE)
        for nb in range(nlt):
            xc_ref[pl.ds(b * spb, spb), nb] = vc4[:, :, nb, :]

        gh = b * spb + jax.lax.broadcasted_iota(jnp.int32, (spb, 1, 1), 0)
        mask = ((gh >= h_lo) & (gh < h_hi)).astype(jnp.float32)
        vm = vc * mask
        vm8 = vm.reshape(-1, SUBLANE, LANE)
        acc_ref[0] += jnp.sum(vm8, axis=0)
        acc_ref[1] += jnp.sum(vm8 * vm8, axis=0)

        @pl.when(b == nb0 - 1)
        def _finalize():
            s1 = jnp.sum(acc_ref[0])
            s2 = jnp.sum(acc_ref[1])
            w = w_ref[0]
            mean_t = w * s1 * inv_cnt
            ex2_t = w * w * s2 * inv_cnt
            var = jnp.maximum(ex2_t - mean_t * mean_t, 0.0)
            a = gamma_ref[0] * jax.lax.rsqrt(var + BN_EPS)
            ab_ref[0] = w * a
            ab_ref[1] = beta_ref[0] - mean_t * a

    @pl.when((p == 1) & (b < nb1))
    def _phase1():
        a = ab_ref[0]
        c = ab_ref[1]
        for i in range(HG):
            h = h_lo + b * HG + i
            col = jnp.concatenate(
                [xc_ref[h, nb, :, :].T for nb in range(nlt)], axis=0)
            o_ref[:, 0, i, :] = jnp.maximum(col * a + c, 0.0)


@functools.partial(jax.jit, static_argnames=("stride", "padding"))
def _forward(x, w_t, gamma, beta, *, stride=1, padding=1):
    N, Cin, H, W = x.shape
    Cin_w, Cout, kH, kW = w_t.shape
    assert Cin == 1 and Cout == 1 and kH == 1 and kW == 1 and stride == 1

    Ho = (H - 1) * stride - 2 * padding + kH
    Wo = (W - 1) * stride - 2 * padding + kW
    assert Ho > 0 and Wo > 0 and Ho % HG == 0
    assert N % LANE == 0 and (W * N) % LANE == 0 and (padding * N) % LANE == 0

    rows_per_slab = W * N // LANE
    crop_rows = Wo * N // LANE
    pad_rows = padding * N // LANE
    nlt = N // LANE

    spb = 1
    for cand in (13, 10, 8, 5, 4, 2):
        if H % cand == 0:
            spb = cand
            break
    nb0 = H // spb
    nb1 = Ho // HG
    nsteps = max(nb0, nb1)

    z = jnp.transpose(x, (2, 3, 1, 0)).reshape(H * rows_per_slab, LANE)
    w1 = w_t.reshape(1).astype(jnp.float32)
    gamma32 = gamma.astype(jnp.float32)
    beta32 = beta.astype(jnp.float32)
    inv_cnt = 1.0 / float(N * Ho * Wo)

    return pl.pallas_call(
        functools.partial(
            _fused_kernel, spb=spb, nb0=nb0, nb1=nb1,
            rows_per_slab=rows_per_slab, crop_rows=crop_rows,
            pad_rows=pad_rows, h_lo=padding, h_hi=padding + Ho,
            nlt=nlt, wo=Wo, inv_cnt=inv_cnt),
        out_shape=jax.ShapeDtypeStruct((N, Cout, Ho, Wo), x.dtype),
        grid=(2, nsteps),
        in_specs=[
            pl.BlockSpec(
                (spb * rows_per_slab, LANE),
                lambda p, b: (jnp.where(p == 0, jnp.minimum(b, nb0 - 1),
                                        nb0 - 1), 0)),
            pl.BlockSpec(memory_space=pltpu.MemorySpace.SMEM),
            pl.BlockSpec(memory_space=pltpu.MemorySpace.SMEM),
            pl.BlockSpec(memory_space=pltpu.MemorySpace.SMEM),
        ],
        out_specs=pl.BlockSpec(
            (N, 1, HG, Wo),
            lambda p, b: (0, 0, jnp.minimum(b, nb1 - 1) * p, 0)),
        scratch_shapes=[
            pltpu.VMEM((H, nlt, Wo, LANE), jnp.float32),
            pltpu.VMEM((2, SUBLANE, LANE), jnp.float32),
            pltpu.SMEM((2,), jnp.float32),
        ],
        compiler_params=pltpu.CompilerParams(
            dimension_semantics=("arbitrary", "arbitrary"),
            vmem_limit_bytes=VMEM_LIMIT),
    )(z, w1, gamma32, beta32)


def kernel(x, w_t, gamma, beta):
    return _forward(x, w_t, gamma, beta, stride=1, padding=1)


# speedup vs baseline: 2.6690x; 1.1598x over previous
import functools

import jax
import jax.numpy as jnp
from jax.experimental import pallas as pl
from jax.experimental.pallas import tpu as pltpu

BN_EPS = 1e-5
LANE = 128
SUBLANE = 8
VMEM_LIMIT = 60 * 1024 * 1024
HG = 16


def _fused_kernel(x_ref, w_ref, gamma_ref, beta_ref, o_ref,
                  xc_ref, acc_ref, ab_ref, *,
                  spb, nb0, nb1, rows_per_slab, crop_rows, pad_rows,
                  h_lo, h_hi, nlt, wo, inv_cnt):
    p = pl.program_id(0)
    b = pl.program_id(1)

    @pl.when((p == 0) & (b < nb0))
    def _phase0():
        @pl.when(b == 0)
        def _():
            acc_ref[...] = jnp.zeros_like(acc_ref)

        v = x_ref[...].reshape(spb, rows_per_slab, LANE)
        vc = v[:, pad_rows:pad_rows + crop_rows, :]
        vc4 = vc.reshape(spb, wo, nlt, LANE)
        for nb in range(nlt):
            xc_ref[pl.ds(b * spb, spb), nb] = vc4[:, :, nb, :]

        gh = b * spb + jax.lax.broadcasted_iota(jnp.int32, (spb, 1, 1), 0)
        mask = ((gh >= h_lo) & (gh < h_hi)).astype(jnp.float32)
        vm = vc * mask
        vm8 = vm.reshape(-1, SUBLANE, LANE)
        acc_ref[0] += jnp.sum(vm8, axis=0)
        acc_ref[1] += jnp.sum(vm8 * vm8, axis=0)

        @pl.when(b == nb0 - 1)
        def _finalize():
            s1 = jnp.sum(acc_ref[0])
            s2 = jnp.sum(acc_ref[1])
            w = w_ref[0]
            mean_t = w * s1 * inv_cnt
            ex2_t = w * w * s2 * inv_cnt
            var = jnp.maximum(ex2_t - mean_t * mean_t, 0.0)
            a = gamma_ref[0] * jax.lax.rsqrt(var + BN_EPS)
            ab_ref[0] = w * a
            ab_ref[1] = beta_ref[0] - mean_t * a

    @pl.when((p == 1) & (b < nb1))
    def _phase1():
        a = ab_ref[0]
        c = ab_ref[1]
        cols = []
        for i in range(HG):
            h = h_lo + b * HG + i
            cols.append(jnp.concatenate(
                [xc_ref[h, nb, :, :].T for nb in range(nlt)], axis=0))
        z = jnp.swapaxes(jnp.stack(cols, axis=0), 0, 1)
        o_ref[:, 0, :, :] = jnp.maximum(z * a + c, 0.0)


@functools.partial(jax.jit, static_argnames=("stride", "padding"))
def _forward(x, w_t, gamma, beta, *, stride=1, padding=1):
    N, Cin, H, W = x.shape
    Cin_w, Cout, kH, kW = w_t.shape
    assert Cin == 1 and Cout == 1 and kH == 1 and kW == 1 and stride == 1

    Ho = (H - 1) * stride - 2 * padding + kH
    Wo = (W - 1) * stride - 2 * padding + kW
    assert Ho > 0 and Wo > 0 and Ho % HG == 0
    assert N % LANE == 0 and (W * N) % LANE == 0 and (padding * N) % LANE == 0

    rows_per_slab = W * N // LANE
    crop_rows = Wo * N // LANE
    pad_rows = padding * N // LANE
    nlt = N // LANE

    spb = 1
    for cand in (13, 10, 8, 5, 4, 2):
        if H % cand == 0:
            spb = cand
            break
    nb0 = H // spb
    nb1 = Ho // HG
    nsteps = max(nb0, nb1)

    z = jnp.transpose(x, (2, 3, 1, 0)).reshape(H * rows_per_slab, LANE)
    w1 = w_t.reshape(1).astype(jnp.float32)
    gamma32 = gamma.astype(jnp.float32)
    beta32 = beta.astype(jnp.float32)
    inv_cnt = 1.0 / float(N * Ho * Wo)

    return pl.pallas_call(
        functools.partial(
            _fused_kernel, spb=spb, nb0=nb0, nb1=nb1,
            rows_per_slab=rows_per_slab, crop_rows=crop_rows,
            pad_rows=pad_rows, h_lo=padding, h_hi=padding + Ho,
            nlt=nlt, wo=Wo, inv_cnt=inv_cnt),
        out_shape=jax.ShapeDtypeStruct((N, Cout, Ho, Wo), x.dtype),
        grid=(2, nsteps),
        in_specs=[
            pl.BlockSpec(
                (spb * rows_per_slab, LANE),
                lambda p, b: (jnp.where(p == 0, jnp.minimum(b, nb0 - 1),
                                        nb0 - 1), 0)),
            pl.BlockSpec(memory_space=pltpu.MemorySpace.SMEM),
            pl.BlockSpec(memory_space=pltpu.MemorySpace.SMEM),
            pl.BlockSpec(memory_space=pltpu.MemorySpace.SMEM),
        ],
        out_specs=pl.BlockSpec(
            (N, 1, HG, Wo),
            lambda p, b: (0, 0, jnp.minimum(b, nb1 - 1) * p, 0)),
        scratch_shapes=[
            pltpu.VMEM((H, nlt, Wo, LANE), jnp.float32),
            pltpu.VMEM((2, SUBLANE, LANE), jnp.float32),
            pltpu.SMEM((2,), jnp.float32),
        ],
        compiler_params=pltpu.CompilerParams(
            dimension_semantics=("arbitrary", "arbitrary"),
            vmem_limit_bytes=VMEM_LIMIT),
    )(z, w1, gamma32, beta32)


def kernel(x, w_t, gamma, beta):
    return _forward(x, w_t, gamma, beta, stride=1, padding=1)
